# Initial kernel scaffold; baseline (speedup 1.0000x reference)
import jax
import jax.numpy as jnp
from jax.experimental import pallas as pl


def kernel(dist_matrix, W_face, W_vertex, W_gluing, W_global):
    raise NotImplementedError("write your pallas kernel here")


# speedup vs baseline: 35.1628x; 35.1628x over previous
import functools

import jax
import jax.numpy as jnp
from jax import lax
from jax.experimental import pallas as pl
from jax.experimental.pallas import tpu as pltpu
from jax.experimental.pallas import tpu_sc as plsc

MAX_D = 513
H = 16
B = 8
N = 512
K = 4
CH = 4
NW = 32
ROWS_PER_TILE = (B * N) // NW
NCHUNK = ROWS_PER_TILE // CH
JBLK = N // 16


def _sc_body(tbl_hbm, dist_hbm, out_hbm, tbl_v, idx_v, out_v, sem):
    nc = 2
    wid = lax.axis_index("s") * nc + lax.axis_index("c")
    b = wid // 4
    ibase = (wid % 4) * ROWS_PER_TILE

    pltpu.sync_copy(tbl_hbm, tbl_v)

    def chunk_body(ci, carry):
        i0 = ibase + ci * CH
        row_off = (b * N + i0) * N
        for k in range(K):
            pltpu.sync_copy(
                dist_hbm.at[k, pl.ds(row_off, CH * N)],
                idx_v.at[pl.ds(k * CH * N, CH * N)])

        def j_body(j, c2):
            for r in range(CH):
                d = [idx_v[pl.ds(k * CH * N + r * N + j * 16, 16)]
                     for k in range(K)]
                for h in range(H):
                    acc = plsc.load_gather(tbl_v, [d[0] + h * MAX_D])
                    acc = acc + plsc.load_gather(
                        tbl_v, [d[1] + (H + h) * MAX_D])
                    acc = acc + plsc.load_gather(
                        tbl_v, [d[2] + (2 * H + h) * MAX_D])
                    acc = acc + plsc.load_gather(
                        tbl_v, [d[3] + (3 * H + h) * MAX_D])
                    out_v[pl.ds(h * (CH * N) + r * N + j * 16, 16)] = acc
            return c2

        lax.fori_loop(0, JBLK, j_body, 0)

        handles = []
        for h in range(H):
            dst_off = ((b * H + h) * N + i0) * N
            handles.append(
                pltpu.async_copy(
                    out_v.at[pl.ds(h * CH * N, CH * N)],
                    out_hbm.at[pl.ds(dst_off, CH * N)],
                    sem,
                ))
        for hd in handles:
            hd.wait()
        return carry

    lax.fori_loop(0, NCHUNK, chunk_body, 0)


@functools.partial(jax.jit)
def _spatial_bias_sc(tbl, dist_t):
    mesh = plsc.VectorSubcoreMesh(core_axis_name="c", subcore_axis_name="s")
    run = functools.partial(
        pl.kernel,
        mesh=mesh,
        out_type=jax.ShapeDtypeStruct((B * H * N * N,), jnp.float32),
        scratch_types=[
            pltpu.VMEM((K * H * MAX_D,), jnp.float32),
            pltpu.VMEM((K * CH * N,), jnp.int32),
            pltpu.VMEM((H * CH * N,), jnp.float32),
            pltpu.SemaphoreType.DMA,
        ],
        compiler_params=pltpu.CompilerParams(needs_layout_passes=False),
    )(_sc_body)
    return run(tbl, dist_t)


def kernel(dist_matrix, W_face, W_vertex, W_gluing, W_global):
    tbl = jnp.stack(
        [W_face.T, W_vertex.T, W_gluing.T, W_global.T]).reshape(-1)
    dist_t = dist_matrix.transpose(3, 0, 1, 2).reshape(K, B * N * N)
    out = _spatial_bias_sc(tbl, dist_t)
    return out.reshape(B, H, N, N)
